# Initial kernel scaffold; baseline (speedup 1.0000x reference)
import jax
import jax.numpy as jnp
from jax.experimental import pallas as pl


def kernel(x, edge_index, batch, params):
    raise NotImplementedError("write your pallas kernel here")


# speedup vs baseline: 3.9254x; 3.9254x over previous
import functools

import jax
import jax.numpy as jnp
from jax import lax
from jax.experimental import pallas as pl
from jax.experimental.pallas import tpu as pltpu
from jax.experimental.pallas import tpu_sc as plsc

N_NODES = 10000
N_EDGES = 160000
D_HID = 512
GROUPS = 64

NCORES = 2
NSUB = 16
LANES = 16

ET = N_EDGES // NSUB
EBATCH = 128
NB = -(-ET // EBATCH)
ET_PAD = NB * EBATCH
ROWS_PER = 624
TAIL_R0 = ROWS_PER * NSUB
TAIL_ROWS = N_NODES - TAIL_R0

BN = 1000
NBLK = N_NODES // BN


def _edge_sum_body(C, *refs):
    zcs = refs[:C]
    src_hbm, dst_hbm = refs[C], refs[C + 1]
    outs = refs[C + 2:2 * C + 2]
    acc, src_t, dst_t, dst2, rows, gsem = refs[2 * C + 2:]

    core = lax.axis_index("c")
    sub = lax.axis_index("s")
    t0 = sub * ET
    r0 = sub * ROWS_PER

    pltpu.sync_copy(src_hbm.at[pl.ds(t0, ET)], src_t.at[pl.ds(0, ET)])
    pltpu.sync_copy(dst_hbm.at[pl.ds(t0, ET)], dst_t.at[pl.ds(0, ET)])
    for i in range(ET, ET_PAD, LANES):
        src_t[pl.ds(i, LANES)] = jnp.zeros((LANES,), jnp.int32)
        dst_t[pl.ds(i, LANES)] = jnp.full((LANES,), N_NODES, jnp.int32)

    def dstcopy(i, _):
        off = pl.multiple_of(i * LANES, LANES)
        b = i // (EBATCH // LANES)
        col = (i % (EBATCH // LANES)) * LANES
        dst2[b, pl.ds(col, LANES)] = dst_t[pl.ds(off, LANES)]
        return 0
    lax.fori_loop(0, ET_PAD // LANES, dstcopy, 0, unroll=8)

    cpc = C // NCORES
    for c in range(C):
        @pl.when(core == c // cpc)
        def _(c=c, first=(c % cpc == 0)):
            z_c = zcs[c]
            out_c = outs[c]
            if not first:
                plsc.subcore_barrier()
            pltpu.sync_copy(z_c.at[pl.ds(r0, ROWS_PER)],
                            acc.at[pl.ds(r0, ROWS_PER)])

            @pl.when(sub == NSUB - 1)
            def _():
                pltpu.sync_copy(z_c.at[pl.ds(TAIL_R0, TAIL_ROWS)],
                                acc.at[pl.ds(TAIL_R0, TAIL_ROWS)])

            plsc.subcore_barrier()

            def ebody(b, _):
                off = pl.multiple_of(b * EBATCH, EBATCH)
                pltpu.async_copy(z_c.at[src_t.at[pl.ds(off, EBATCH)]],
                                 rows, gsem).wait()
                pltpu.sync_copy(rows, acc.at[dst2.at[b]], add=True)
                return 0
            lax.fori_loop(0, NB, ebody, 0)

            plsc.subcore_barrier()
            pltpu.sync_copy(acc.at[pl.ds(r0, ROWS_PER)],
                            out_c.at[pl.ds(r0, ROWS_PER)])

            @pl.when(sub == NSUB - 1)
            def _():
                pltpu.sync_copy(acc.at[pl.ds(TAIL_R0, TAIL_ROWS)],
                                out_c.at[pl.ds(TAIL_R0, TAIL_ROWS)])


@functools.cache
def _edge_sum(C):
    return pl.kernel(
        functools.partial(_edge_sum_body, C),
        out_type=[jax.ShapeDtypeStruct((N_NODES, 128), jnp.float32)] * C,
        mesh=plsc.VectorSubcoreMesh(core_axis_name="c", subcore_axis_name="s",
                                    num_cores=NCORES, num_subcores=NSUB),
        scratch_types=[
            pltpu.VMEM_SHARED((N_NODES + LANES, 128), jnp.float32),
            pltpu.VMEM((ET_PAD,), jnp.int32),
            pltpu.VMEM((ET_PAD,), jnp.int32),
            pltpu.VMEM((NB, EBATCH), jnp.int32),
            pltpu.VMEM((EBATCH, 128), jnp.float32),
            pltpu.SemaphoreType.DMA,
        ],
    )


def _mm1_body(C, *refs):
    s_refs = refs[:C]
    w_ref, b_ref, g_ref, be_ref, h_ref, st_ref = refs[C:]
    i = pl.program_id(0)
    agg = jnp.concatenate([s_refs[c][...] for c in range(C)], axis=-1)
    h = jnp.dot(agg, w_ref[...], preferred_element_type=jnp.float32) + b_ref[...]
    h_ref[...] = h

    @pl.when(i == 0)
    def _():
        st_ref[...] = jnp.zeros_like(st_ref)

    st_ref[0:1, :] += jnp.sum(h, axis=0, keepdims=True)
    st_ref[1:2, :] += jnp.sum(h * h, axis=0, keepdims=True)

    @pl.when(i == pl.num_programs(0) - 1)
    def _():
        mean = st_ref[0:1, :] / N_NODES
        var = st_ref[1:2, :] / N_NODES - mean * mean
        a = g_ref[...] * lax.rsqrt(var + 1e-5)
        st_ref[2:3, :] = a
        st_ref[3:4, :] = be_ref[...] - mean * a


@functools.cache
def _mm1(C):
    d_in = C * 128
    return pl.pallas_call(
        functools.partial(_mm1_body, C),
        grid=(NBLK,),
        in_specs=[pl.BlockSpec((BN, 128), lambda i: (i, 0))] * C + [
            pl.BlockSpec((d_in, D_HID), lambda i: (0, 0)),
            pl.BlockSpec((1, D_HID), lambda i: (0, 0)),
            pl.BlockSpec((1, D_HID), lambda i: (0, 0)),
            pl.BlockSpec((1, D_HID), lambda i: (0, 0)),
        ],
        out_specs=[
            pl.BlockSpec((BN, D_HID), lambda i: (i, 0)),
            pl.BlockSpec((8, D_HID), lambda i: (0, 0)),
        ],
        out_shape=[
            jax.ShapeDtypeStruct((N_NODES, D_HID), jnp.float32),
            jax.ShapeDtypeStruct((8, D_HID), jnp.float32),
        ],
    )


def _mm2_body(h_ref, st_ref, w_ref, b2_ref, *z_refs):
    h2 = jnp.maximum(h_ref[...] * st_ref[2:3, :] + st_ref[3:4, :], 0.0)
    z = jnp.dot(h2, w_ref[...], preferred_element_type=jnp.float32) + b2_ref[...]
    z = jnp.maximum(z, 0.0)
    for c in range(4):
        z_refs[c][...] = z[:, c * 128:(c + 1) * 128]


_mm2 = pl.pallas_call(
    _mm2_body,
    grid=(NBLK,),
    in_specs=[
        pl.BlockSpec((BN, D_HID), lambda i: (i, 0)),
        pl.BlockSpec((8, D_HID), lambda i: (0, 0)),
        pl.BlockSpec((D_HID, D_HID), lambda i: (0, 0)),
        pl.BlockSpec((1, D_HID), lambda i: (0, 0)),
    ],
    out_specs=[pl.BlockSpec((BN, 128), lambda i: (i, 0))] * 4,
    out_shape=[jax.ShapeDtypeStruct((N_NODES, 128), jnp.float32)] * 4,
)


def _mm2p_body(h_ref, st_ref, w_ref, b2_ref, bt_ref, z_ref, p_ref):
    i = pl.program_id(0)
    h2 = jnp.maximum(h_ref[...] * st_ref[2:3, :] + st_ref[3:4, :], 0.0)
    z = jnp.dot(h2, w_ref[...], preferred_element_type=jnp.float32) + b2_ref[...]
    z = jnp.maximum(z, 0.0)
    z_ref[...] = z

    oh = (lax.broadcasted_iota(jnp.int32, (GROUPS, BN), 0)
          == bt_ref[0, 0, :][None, :]).astype(jnp.float32)

    @pl.when(i == 0)
    def _():
        p_ref[...] = jnp.zeros_like(p_ref)

    p_ref[...] += jnp.dot(oh, z, preferred_element_type=jnp.float32)


_mm2p = pl.pallas_call(
    _mm2p_body,
    grid=(NBLK,),
    in_specs=[
        pl.BlockSpec((BN, D_HID), lambda i: (i, 0)),
        pl.BlockSpec((8, D_HID), lambda i: (0, 0)),
        pl.BlockSpec((D_HID, D_HID), lambda i: (0, 0)),
        pl.BlockSpec((1, D_HID), lambda i: (0, 0)),
        pl.BlockSpec((1, 1, BN), lambda i: (i, 0, 0)),
    ],
    out_specs=[
        pl.BlockSpec((BN, D_HID), lambda i: (i, 0)),
        pl.BlockSpec((GROUPS, D_HID), lambda i: (0, 0)),
    ],
    out_shape=[
        jax.ShapeDtypeStruct((N_NODES, D_HID), jnp.float32),
        jax.ShapeDtypeStruct((GROUPS, D_HID), jnp.float32),
    ],
)


def kernel(x, edge_index, batch, params):
    src = edge_index[0]
    dst = edge_index[1]
    batch3d = batch.reshape(NBLK, 1, BN)

    zcs = [x[:, 0:128], x[:, 128:256]]

    n_layers = len(params)
    for l, p in enumerate(params):
        C = len(zcs)
        scs = _edge_sum(C)(*zcs, src, dst)
        h, st = _mm1(C)(*scs, p['W1'], p['b1'].reshape(1, -1),
                        p['gamma'].reshape(1, -1), p['beta'].reshape(1, -1))
        if l + 1 < n_layers:
            zcs = _mm2(h, st, p['W2'], p['b2'].reshape(1, -1))
        else:
            z, pool = _mm2p(h, st, p['W2'], p['b2'].reshape(1, -1), batch3d)
    return z, pool
